# baseline (speedup 1.0000x reference)
import functools

import jax
import jax.numpy as jnp
from jax import lax
from jax.experimental import pallas as pl
from jax.experimental.pallas import tpu as pltpu
from jax.experimental.pallas import tpu_sc as plsc

B = 16384
E = 1000000
R = 1000
D = 64
L = 16
NC = 2
NS = 16
TPW = B // NS
NG = D // (8 * NC)
CW = 71424
NCHUNK = 14
TINY = E - NCHUNK * CW
NLOAD = 6
CSL = CW // NLOAD
MARGIN = 1.0


def _sc_body(heads2d, rels2d, tails2d, entT, relT, proT, rpT,
             pos_out, neg_out,
             buf0, buf1, bufT, h_idx, r_idx, t_idx, h_loc, t_loc,
             h_inb, t_inb, h_tmp, t_tmp, h_vals, t_vals, rel8,
             pos_acc, neg_acc, semL, semG, semT):
    core = lax.axis_index("c")
    s = lax.axis_index("s")

    pltpu.sync_copy(heads2d.at[pl.ds(s * 8, 8)], h_idx)
    pltpu.sync_copy(rels2d.at[pl.ds(s * 8, 8)], r_idx)
    pltpu.sync_copy(tails2d.at[pl.ds(s * 8, 8)], t_idx)

    def zero1(ref):
        def body(g, _):
            ref[pl.ds(g * L, L)] = jnp.zeros((L,), jnp.float32)
            return 0
        lax.fori_loop(0, TPW // L, body, 0, unroll=False)

    def zero2(ref):
        def body(m, _):
            ref[m >> 6, pl.ds(jnp.bitwise_and(m, 63) * L, L)] = (
                jnp.zeros((L,), jnp.float32))
            return 0
        lax.fori_loop(0, 8 * (TPW // L), body, 0, unroll=False)

    def load_chunk(tab, row0, lo, buf):
        @pl.when(s < NLOAD)
        def _():
            col = pl.multiple_of(lo + s * CSL, 128)
            scol = pl.multiple_of(s * CSL, 128)
            pltpu.async_copy(tab.at[pl.ds(row0, 8), pl.ds(col, CSL)],
                             buf.at[:, pl.ds(scol, CSL)], semL)

    def wait_load(tab, row0, buf):
        @pl.when(s < NLOAD)
        def _():
            pltpu.make_async_copy(
                tab.at[pl.ds(row0, 8), pl.ds(0, CSL)],
                buf.at[:, pl.ds(0, CSL)], semL).wait()

    def prep_chunk(lo, cw):
        def body(g, _):
            r0 = g >> 3
            csl = pl.ds(jnp.bitwise_and(g, 7) * L, L)
            gsl = pl.ds(g * L, L)
            for idx, loc, inb in ((h_idx, h_loc, h_inb),
                                  (t_idx, t_loc, t_inb)):
                v = idx[r0, csl]
                rel = v - lo
                loc[r0, csl] = jnp.clip(rel, 0, cw - 1)
                ok = jnp.logical_and(rel >= 0, rel < cw)
                inb[gsl] = jnp.where(ok, 1.0, 0.0).astype(jnp.float32)
            return 0
        lax.fori_loop(0, TPW // L, body, 0, unroll=False)

    def gather_chunk(buf, dummy):
        def body(f, _):
            for j in range(TPW // 128):
                dst = pl.ds(j * 128, 128)
                pltpu.async_copy(buf.at[f].at[h_loc.at[j]],
                                 h_tmp.at[f, dst], semG)
                pltpu.async_copy(buf.at[f].at[t_loc.at[j]],
                                 t_tmp.at[f, dst], semG)
            return 0
        lax.fori_loop(0, 8, body, 0, unroll=False)
        pltpu.make_async_copy(dummy, h_tmp, semG).wait()
        pltpu.make_async_copy(dummy, t_tmp, semG).wait()

    def handle_chunk(tab, row0, k, buf, dummy, fire_k, fire_buf):
        wait_load(tab, row0, buf)
        plsc.subcore_barrier()
        prep_chunk(k * CW, CW)
        gather_chunk(buf, dummy)
        plsc.subcore_barrier()
        if fire_k is not None:
            load_chunk(tab, row0, fire_k * CW, fire_buf)
        merge_chunk()

    def handle_tiny(dummy):
        plsc.subcore_barrier()
        prep_chunk(NCHUNK * CW, TINY)
        gather_chunk(bufT, dummy)
        merge_chunk()

    def merge_chunk():
        def body(m, _):
            f = m >> 6
            g = jnp.bitwise_and(m, 63)
            gsl = pl.ds(g * L, L)
            h_vals[f, gsl] = h_vals[f, gsl] + h_inb[gsl] * h_tmp[f, gsl]
            t_vals[f, gsl] = t_vals[f, gsl] + t_inb[gsl] * t_tmp[f, gsl]
            return 0
        lax.fori_loop(0, 8 * (TPW // L), body, 0, unroll=False)

    def accumulate(acc):
        def body(m, _):
            f = m >> 6
            g = jnp.bitwise_and(m, 63)
            gsl = pl.ds(g * L, L)
            r16 = r_idx[g >> 3, pl.ds(jnp.bitwise_and(g, 7) * L, L)]
            ffull = jnp.full((L,), 0, jnp.int32) + f
            rv = plsc.load_gather(rel8, [ffull, r16])
            d = h_vals[f, gsl] + rv - t_vals[f, gsl]
            acc[gsl] = acc[gsl] + d * d
            return 0
        lax.fori_loop(0, 8 * (TPW // L), body, 0, unroll=False)

    for tab, rtab, acc, out in ((entT, relT, pos_acc, pos_out),
                                (proT, rpT, neg_acc, neg_out)):
        dummy = tab.at[pl.ds(0, 8), pl.ds(0, TPW)]
        zero1(acc)

        def group_body(gix, _):
            gi = core * NG + gix
            row0 = pl.multiple_of(gi * 8, 8)
            zero2(h_vals)
            zero2(t_vals)
            pltpu.sync_copy(rtab.at[pl.ds(row0, 8)], rel8)

            @pl.when(s == 0)
            def _():
                pltpu.async_copy(
                    tab.at[pl.ds(row0, 8), pl.ds(NCHUNK * CW, TINY)],
                    bufT, semT).wait()

            load_chunk(tab, row0, 0, buf0)
            load_chunk(tab, row0, CW, buf1)

            def pair(k2, _):
                k0 = k2 * 2
                handle_chunk(tab, row0, k0, buf0, dummy, k0 + 2, buf0)
                handle_chunk(tab, row0, k0 + 1, buf1, dummy, k0 + 3, buf1)
                return 0

            lax.fori_loop(0, (NCHUNK - 2) // 2, pair, 0, unroll=False)
            handle_chunk(tab, row0, NCHUNK - 2, buf0, dummy, None, None)
            handle_chunk(tab, row0, NCHUNK - 1, buf1, dummy, None, None)
            handle_tiny(dummy)
            accumulate(acc)
            return 0

        lax.fori_loop(0, NG, group_body, 0, unroll=False)
        pltpu.sync_copy(acc, out.at[core, pl.ds(s * TPW, TPW)])


@functools.cache
def _sc_call():
    mesh = plsc.VectorSubcoreMesh(core_axis_name="c", subcore_axis_name="s",
                                  num_cores=NC, num_subcores=NS)
    return pl.kernel(
        _sc_body,
        out_type=(jax.ShapeDtypeStruct((NC, B), jnp.float32),
                  jax.ShapeDtypeStruct((NC, B), jnp.float32)),
        mesh=mesh,
        scratch_types=[
            pltpu.VMEM_SHARED((8, CW), jnp.float32),
            pltpu.VMEM_SHARED((8, CW), jnp.float32),
            pltpu.VMEM_SHARED((8, TINY), jnp.float32),
            pltpu.VMEM((8, 128), jnp.int32),
            pltpu.VMEM((8, 128), jnp.int32),
            pltpu.VMEM((8, 128), jnp.int32),
            pltpu.VMEM((8, 128), jnp.int32),
            pltpu.VMEM((8, 128), jnp.int32),
            pltpu.VMEM((TPW,), jnp.float32),
            pltpu.VMEM((TPW,), jnp.float32),
            pltpu.VMEM((8, TPW), jnp.float32),
            pltpu.VMEM((8, TPW), jnp.float32),
            pltpu.VMEM((8, TPW), jnp.float32),
            pltpu.VMEM((8, TPW), jnp.float32),
            pltpu.VMEM((8, R), jnp.float32),
            pltpu.VMEM((TPW,), jnp.float32),
            pltpu.VMEM((TPW,), jnp.float32),
            pltpu.SemaphoreType.DMA,
            pltpu.SemaphoreType.DMA,
            pltpu.SemaphoreType.DMA,
        ],
        compiler_params=pltpu.CompilerParams(use_tc_tiling_on_sc=False,
                                             needs_layout_passes=False),
    )


def _tc_body(pos_ref, neg_ref, out_ref):
    p = jnp.sqrt(pos_ref[0] + pos_ref[1])
    n = jnp.sqrt(neg_ref[0] + neg_ref[1])
    out_ref[0, 0] = jnp.sum(jnp.maximum(p - n + MARGIN, 0.0)) * (1.0 / B)


_tc_call = pl.pallas_call(
    _tc_body,
    out_shape=jax.ShapeDtypeStruct((1, 1), jnp.float32),
    in_specs=[pl.BlockSpec(memory_space=pltpu.VMEM),
              pl.BlockSpec(memory_space=pltpu.VMEM)],
    out_specs=pl.BlockSpec(memory_space=pltpu.SMEM),
)


def kernel(heads, relations, tails, entity_embedding, relation_embedding,
           entity_projection, relation_projection):
    heads2d = heads.reshape(B // 128, 128)
    rels2d = relations.reshape(B // 128, 128)
    tails2d = tails.reshape(B // 128, 128)
    pos_sq, neg_sq = _sc_call()(heads2d, rels2d, tails2d,
                                entity_embedding.T, relation_embedding.T,
                                entity_projection.T, relation_projection.T)
    loss = _tc_call(pos_sq.reshape(NC, 128, B // 128),
                    neg_sq.reshape(NC, 128, B // 128))
    return loss[0, 0]


# speedup vs baseline: 20.9137x; 20.9137x over previous
import functools

import jax
import jax.numpy as jnp
from jax import lax
from jax.experimental import pallas as pl
from jax.experimental.pallas import tpu as pltpu
from jax.experimental.pallas import tpu_sc as plsc

B = 16384
D = 64
L = 16
NC = 2
NS = 16
NW = NC * NS
BPW = B // NW
CH = 128
NCHUNK = BPW // CH
MARGIN = 1.0


def _row_sq_norms(h_rows, r_rows, t_rows, sq_v):
    lanes = lax.iota(jnp.int32, L)

    def lane_sum(v):
        for sh in (8, 4, 2, 1):
            idx = jnp.bitwise_and(lanes + sh, L - 1)
            v = v + v.at[idx].get(mode="promise_in_bounds")
        return v

    def body(g, _):
        vec = jnp.zeros((L,), jnp.float32)
        for j in range(L):
            i = g * L + j
            acc = jnp.zeros((L,), jnp.float32)
            for c in range(D // L):
                sl = pl.ds(c * L, L)
                d = h_rows[i, sl] + r_rows[i, sl] - t_rows[i, sl]
                acc = acc + d * d
            vec = jnp.where(lanes == j, lane_sum(acc), vec)
        sq_v[pl.ds(g * L, L)] = vec
        return 0

    lax.fori_loop(0, BPW // L, body, 0, unroll=False)


def _sc_body(heads2d, rels2d, tails2d, ent_tab, rel_tab, out,
             h_idx, r_idx, t_idx, h_rows, r_rows, t_rows, sq_v, sem):
    wid = lax.axis_index("s") * NC + lax.axis_index("c")
    base_row = wid * NCHUNK

    pltpu.sync_copy(heads2d.at[pl.ds(base_row, NCHUNK)], h_idx)
    pltpu.sync_copy(rels2d.at[pl.ds(base_row, NCHUNK)], r_idx)
    pltpu.sync_copy(tails2d.at[pl.ds(base_row, NCHUNK)], t_idx)

    descs = []
    for j in range(NCHUNK):
        dst = pl.ds(j * CH, CH)
        descs.append(pltpu.async_copy(ent_tab.at[h_idx.at[j]],
                                      h_rows.at[dst], sem))
        descs.append(pltpu.async_copy(rel_tab.at[r_idx.at[j]],
                                      r_rows.at[dst], sem))
        descs.append(pltpu.async_copy(ent_tab.at[t_idx.at[j]],
                                      t_rows.at[dst], sem))
    for desc in descs:
        desc.wait()
    _row_sq_norms(h_rows, r_rows, t_rows, sq_v)
    pltpu.sync_copy(sq_v, out.at[pl.ds(wid * BPW, BPW)])


@functools.cache
def _sc_call():
    mesh = plsc.VectorSubcoreMesh(core_axis_name="c", subcore_axis_name="s",
                                  num_cores=NC, num_subcores=NS)
    return pl.kernel(
        _sc_body,
        out_type=jax.ShapeDtypeStruct((B,), jnp.float32),
        mesh=mesh,
        scratch_types=[
            pltpu.VMEM((NCHUNK, CH), jnp.int32),
            pltpu.VMEM((NCHUNK, CH), jnp.int32),
            pltpu.VMEM((NCHUNK, CH), jnp.int32),
            pltpu.VMEM((BPW, D), jnp.float32),
            pltpu.VMEM((BPW, D), jnp.float32),
            pltpu.VMEM((BPW, D), jnp.float32),
            pltpu.VMEM((BPW,), jnp.float32),
            pltpu.SemaphoreType.DMA,
        ],
        compiler_params=pltpu.CompilerParams(use_tc_tiling_on_sc=False),
    )


def _tc_body(pos_ref, neg_ref, out_ref):
    p = jnp.sqrt(pos_ref[...])
    n = jnp.sqrt(neg_ref[...])
    out_ref[0, 0] = jnp.sum(jnp.maximum(p - n + MARGIN, 0.0)) * (1.0 / B)


_tc_call = pl.pallas_call(
    _tc_body,
    out_shape=jax.ShapeDtypeStruct((1, 1), jnp.float32),
    in_specs=[pl.BlockSpec(memory_space=pltpu.VMEM),
              pl.BlockSpec(memory_space=pltpu.VMEM)],
    out_specs=pl.BlockSpec(memory_space=pltpu.SMEM),
)


def kernel(heads, relations, tails, entity_embedding, relation_embedding,
           entity_projection, relation_projection):
    heads2d = heads.reshape(B // CH, CH)
    rels2d = relations.reshape(B // CH, CH)
    tails2d = tails.reshape(B // CH, CH)
    pos_sq = _sc_call()(heads2d, rels2d, tails2d,
                        entity_embedding, relation_embedding)
    neg_sq = _sc_call()(heads2d, rels2d, tails2d,
                        entity_projection, relation_projection)
    loss = _tc_call(pos_sq.reshape(CH, B // CH), neg_sq.reshape(CH, B // CH))
    return loss[0, 0]


# speedup vs baseline: 26.2742x; 1.2563x over previous
import functools

import jax
import jax.numpy as jnp
from jax import lax
from jax.experimental import pallas as pl
from jax.experimental.pallas import tpu as pltpu
from jax.experimental.pallas import tpu_sc as plsc

B = 16384
D = 64
L = 16
NC = 2
NS = 16
NW = NC * NS
BPW = B // NW
CH = 128
NCHUNK = BPW // CH
MARGIN = 1.0


def _row_sq_norms(h_rows, r_rows, t_rows, sq_v):
    lanes = lax.iota(jnp.int32, L)

    def lane_sum(v):
        for sh in (8, 4, 2, 1):
            idx = jnp.bitwise_and(lanes + sh, L - 1)
            v = v + v.at[idx].get(mode="promise_in_bounds")
        return v

    def body(g, _):
        vec = jnp.zeros((L,), jnp.float32)
        for j in range(L):
            i = g * L + j
            acc = jnp.zeros((L,), jnp.float32)
            for c in range(D // L):
                sl = pl.ds(c * L, L)
                d = h_rows[i, sl] + r_rows[i, sl] - t_rows[i, sl]
                acc = acc + d * d
            vec = jnp.where(lanes == j, lane_sum(acc), vec)
        sq_v[pl.ds(g * L, L)] = vec
        return 0

    lax.fori_loop(0, BPW // L, body, 0, unroll=False)


def _sc_body(heads2d, rels2d, tails2d, ent_tab, rel_tab, out,
             h_idx, r_idx, t_idx, h_rows, r_rows, t_rows, sq_v, sem):
    wid = lax.axis_index("s") * NC + lax.axis_index("c")
    base_row = wid * NCHUNK

    pltpu.sync_copy(heads2d.at[pl.ds(base_row, NCHUNK)], h_idx)
    pltpu.sync_copy(rels2d.at[pl.ds(base_row, NCHUNK)], r_idx)
    pltpu.sync_copy(tails2d.at[pl.ds(base_row, NCHUNK)], t_idx)

    descs = []
    for j in range(NCHUNK):
        dst = pl.ds(j * CH, CH)
        descs.append(pltpu.async_copy(ent_tab.at[h_idx.at[j]],
                                      h_rows.at[dst], sem))
        descs.append(pltpu.async_copy(rel_tab.at[r_idx.at[j]],
                                      r_rows.at[dst], sem))
        descs.append(pltpu.async_copy(ent_tab.at[t_idx.at[j]],
                                      t_rows.at[dst], sem))
    for desc in descs:
        desc.wait()
    _row_sq_norms(h_rows, r_rows, t_rows, sq_v)
    pltpu.sync_copy(sq_v, out.at[pl.ds(wid * BPW, BPW)])


@functools.cache
def _sc_call():
    mesh = plsc.VectorSubcoreMesh(core_axis_name="c", subcore_axis_name="s",
                                  num_cores=NC, num_subcores=NS)
    return pl.kernel(
        _sc_body,
        out_type=jax.ShapeDtypeStruct((B,), jnp.float32),
        mesh=mesh,
        scratch_types=[
            pltpu.VMEM((NCHUNK, CH), jnp.int32),
            pltpu.VMEM((NCHUNK, CH), jnp.int32),
            pltpu.VMEM((NCHUNK, CH), jnp.int32),
            pltpu.VMEM((BPW, D), jnp.float32),
            pltpu.VMEM((BPW, D), jnp.float32),
            pltpu.VMEM((BPW, D), jnp.float32),
            pltpu.VMEM((BPW,), jnp.float32),
            pltpu.SemaphoreType.DMA,
        ],
        compiler_params=pltpu.CompilerParams(use_tc_tiling_on_sc=False),
    )


HALF = 256


def _sc_body_rowdma(heads2d, rels2d, tails2d, ent_tab, rel_tab, out,
                    h_idx, r_idx, t_idx, h_rows, r_rows, t_rows, sq_v, sem):
    wid = lax.axis_index("s") * NC + lax.axis_index("c")
    rows_per_half = HALF // CH

    for k in range(BPW // HALF):
        base_row = wid * (BPW // CH) + k * rows_per_half
        sl = pl.ds(base_row, rows_per_half)
        pltpu.sync_copy(heads2d.at[sl], h_idx)
        pltpu.sync_copy(rels2d.at[sl], r_idx)
        pltpu.sync_copy(tails2d.at[sl], t_idx)

        def enq(g, _):
            r0 = g >> 3
            c0 = jnp.bitwise_and(g, (CH // L) - 1) * L
            hv = h_idx[r0, pl.ds(c0, L)]
            rv = r_idx[r0, pl.ds(c0, L)]
            tv = t_idx[r0, pl.ds(c0, L)]
            for j in range(L):
                i = g * L + j
                pltpu.async_copy(ent_tab.at[hv[j]], h_rows.at[i], sem)
                pltpu.async_copy(rel_tab.at[rv[j]], r_rows.at[i], sem)
                pltpu.async_copy(ent_tab.at[tv[j]], t_rows.at[i], sem)
            return 0

        lax.fori_loop(0, HALF // L, enq, 0, unroll=False)
        pltpu.make_async_copy(ent_tab.at[pl.ds(0, HALF)], h_rows, sem).wait()
        pltpu.make_async_copy(ent_tab.at[pl.ds(0, HALF)], r_rows, sem).wait()
        pltpu.make_async_copy(ent_tab.at[pl.ds(0, HALF)], t_rows, sem).wait()
        _row_sq_norms_half(h_rows, r_rows, t_rows, sq_v, k * HALF)
    pltpu.sync_copy(sq_v, out.at[pl.ds(wid * BPW, BPW)])


def _row_sq_norms_half(h_rows, r_rows, t_rows, sq_v, base):
    lanes = lax.iota(jnp.int32, L)

    def lane_sum(v):
        for sh in (8, 4, 2, 1):
            idx = jnp.bitwise_and(lanes + sh, L - 1)
            v = v + v.at[idx].get(mode="promise_in_bounds")
        return v

    def body(g, _):
        vec = jnp.zeros((L,), jnp.float32)
        for j in range(L):
            i = g * L + j
            acc = jnp.zeros((L,), jnp.float32)
            for c in range(D // L):
                sl = pl.ds(c * L, L)
                d = h_rows[i, sl] + r_rows[i, sl] - t_rows[i, sl]
                acc = acc + d * d
            vec = jnp.where(lanes == j, lane_sum(acc), vec)
        sq_v[pl.ds(base + g * L, L)] = vec
        return 0

    lax.fori_loop(0, HALF // L, body, 0, unroll=False)


@functools.cache
def _sc_call_rowdma():
    mesh = plsc.VectorSubcoreMesh(core_axis_name="c", subcore_axis_name="s",
                                  num_cores=NC, num_subcores=NS)
    return pl.kernel(
        _sc_body_rowdma,
        out_type=jax.ShapeDtypeStruct((B,), jnp.float32),
        mesh=mesh,
        scratch_types=[
            pltpu.VMEM((HALF // CH, CH), jnp.int32),
            pltpu.VMEM((HALF // CH, CH), jnp.int32),
            pltpu.VMEM((HALF // CH, CH), jnp.int32),
            pltpu.VMEM((HALF, D), jnp.float32),
            pltpu.VMEM((HALF, D), jnp.float32),
            pltpu.VMEM((HALF, D), jnp.float32),
            pltpu.VMEM((BPW,), jnp.float32),
            pltpu.SemaphoreType.DMA,
        ],
        compiler_params=pltpu.CompilerParams(use_tc_tiling_on_sc=True),
    )


def _tc_body(pos_ref, neg_ref, out_ref):
    p = jnp.sqrt(pos_ref[...])
    n = jnp.sqrt(neg_ref[...])
    out_ref[0, 0] = jnp.sum(jnp.maximum(p - n + MARGIN, 0.0)) * (1.0 / B)


_tc_call = pl.pallas_call(
    _tc_body,
    out_shape=jax.ShapeDtypeStruct((1, 1), jnp.float32),
    in_specs=[pl.BlockSpec(memory_space=pltpu.VMEM),
              pl.BlockSpec(memory_space=pltpu.VMEM)],
    out_specs=pl.BlockSpec(memory_space=pltpu.SMEM),
)


def kernel(heads, relations, tails, entity_embedding, relation_embedding,
           entity_projection, relation_projection):
    heads2d = heads.reshape(B // CH, CH)
    rels2d = relations.reshape(B // CH, CH)
    tails2d = tails.reshape(B // CH, CH)
    pos_sq = _sc_call()(heads2d, rels2d, tails2d,
                        entity_embedding, relation_embedding)
    neg_sq = _sc_call_rowdma()(heads2d, rels2d, tails2d,
                               entity_projection, relation_projection)
    loss = _tc_call(pos_sq.reshape(CH, B // CH), neg_sq.reshape(CH, B // CH))
    return loss[0, 0]


# speedup vs baseline: 31.6767x; 1.2056x over previous
import functools

import jax
import jax.numpy as jnp
from jax import lax
from jax.experimental import pallas as pl
from jax.experimental.pallas import tpu as pltpu
from jax.experimental.pallas import tpu_sc as plsc

B = 16384
D = 64
L = 16
NC = 2
NS = 16
NW = NC * NS
BPW = B // NW
CH = 128
HALF = 256
MARGIN = 1.0


def _row_sq_norms(h_rows, r_rows, t_rows, sq_v, base):
    lanes = lax.iota(jnp.int32, L)

    def lane_sum(v):
        for sh in (8, 4, 2, 1):
            idx = jnp.bitwise_and(lanes + sh, L - 1)
            v = v + v.at[idx].get(mode="promise_in_bounds")
        return v

    def body(g, _):
        vec = jnp.zeros((L,), jnp.float32)
        for j in range(L):
            i = g * L + j
            acc = jnp.zeros((L,), jnp.float32)
            for c in range(D // L):
                sl = pl.ds(c * L, L)
                d = h_rows[i, sl] + r_rows[i, sl] - t_rows[i, sl]
                acc = acc + d * d
            vec = jnp.where(lanes == j, lane_sum(acc), vec)
        sq_v[pl.ds(base + g * L, L)] = vec
        return 0

    lax.fori_loop(0, HALF // L, body, 0, unroll=False)


def _sc_body(heads2d, rels2d, tails2d, ent_emb, rel_emb, ent_proj, rel_proj,
             pos_out, neg_out,
             h_idx, r_idx, t_idx, h_rows, r_rows, t_rows, sq_v, sem):
    wid = lax.axis_index("s") * NC + lax.axis_index("c")
    rows_per_half = HALF // CH

    for ent_tab, rel_tab, out in ((ent_emb, rel_emb, pos_out),
                                  (ent_proj, rel_proj, neg_out)):
        for k in range(BPW // HALF):
            base_row = wid * (BPW // CH) + k * rows_per_half
            sl = pl.ds(base_row, rows_per_half)
            pltpu.sync_copy(heads2d.at[sl], h_idx)
            pltpu.sync_copy(rels2d.at[sl], r_idx)
            pltpu.sync_copy(tails2d.at[sl], t_idx)

            def enq(g, _):
                r0 = g >> 3
                c0 = jnp.bitwise_and(g, (CH // L) - 1) * L
                hv = h_idx[r0, pl.ds(c0, L)]
                rv = r_idx[r0, pl.ds(c0, L)]
                tv = t_idx[r0, pl.ds(c0, L)]
                for j in range(L):
                    i = g * L + j
                    pltpu.async_copy(ent_tab.at[hv[j]], h_rows.at[i], sem)
                    pltpu.async_copy(rel_tab.at[rv[j]], r_rows.at[i], sem)
                    pltpu.async_copy(ent_tab.at[tv[j]], t_rows.at[i], sem)
                return 0

            lax.fori_loop(0, HALF // L, enq, 0, unroll=False)
            pltpu.make_async_copy(ent_emb.at[pl.ds(0, HALF)], h_rows, sem).wait()
            pltpu.make_async_copy(ent_emb.at[pl.ds(0, HALF)], r_rows, sem).wait()
            pltpu.make_async_copy(ent_emb.at[pl.ds(0, HALF)], t_rows, sem).wait()

            _row_sq_norms(h_rows, r_rows, t_rows, sq_v, k * HALF)
        pltpu.sync_copy(sq_v, out.at[pl.ds(wid * BPW, BPW)])


@functools.cache
def _sc_call():
    mesh = plsc.VectorSubcoreMesh(core_axis_name="c", subcore_axis_name="s",
                                  num_cores=NC, num_subcores=NS)
    return pl.kernel(
        _sc_body,
        out_type=(jax.ShapeDtypeStruct((B,), jnp.float32),
                  jax.ShapeDtypeStruct((B,), jnp.float32)),
        mesh=mesh,
        scratch_types=[
            pltpu.VMEM((HALF // CH, CH), jnp.int32),
            pltpu.VMEM((HALF // CH, CH), jnp.int32),
            pltpu.VMEM((HALF // CH, CH), jnp.int32),
            pltpu.VMEM((HALF, D), jnp.float32),
            pltpu.VMEM((HALF, D), jnp.float32),
            pltpu.VMEM((HALF, D), jnp.float32),
            pltpu.VMEM((BPW,), jnp.float32),
            pltpu.SemaphoreType.DMA,
        ],
        compiler_params=pltpu.CompilerParams(use_tc_tiling_on_sc=True),
    )


def _tc_body(pos_ref, neg_ref, out_ref):
    p = jnp.sqrt(pos_ref[...])
    n = jnp.sqrt(neg_ref[...])
    out_ref[0, 0] = jnp.sum(jnp.maximum(p - n + MARGIN, 0.0)) * (1.0 / B)


_tc_call = pl.pallas_call(
    _tc_body,
    out_shape=jax.ShapeDtypeStruct((1, 1), jnp.float32),
    in_specs=[pl.BlockSpec(memory_space=pltpu.VMEM),
              pl.BlockSpec(memory_space=pltpu.VMEM)],
    out_specs=pl.BlockSpec(memory_space=pltpu.SMEM),
)


def kernel(heads, relations, tails, entity_embedding, relation_embedding,
           entity_projection, relation_projection):
    heads2d = heads.reshape(B // CH, CH)
    rels2d = relations.reshape(B // CH, CH)
    tails2d = tails.reshape(B // CH, CH)
    pos_sq, neg_sq = _sc_call()(heads2d, rels2d, tails2d,
                                entity_embedding, relation_embedding,
                                entity_projection, relation_projection)
    loss = _tc_call(pos_sq.reshape(CH, B // CH), neg_sq.reshape(CH, B // CH))
    return loss[0, 0]
